# baseline (speedup 1.0000x reference)
import functools

import jax
import jax.numpy as jnp
from jax import lax
from jax.experimental import pallas as pl
from jax.experimental.pallas import tpu as pltpu
from jax.experimental.pallas import tpu_sc as plsc

N = 10000
E = 320000
C = 1000
IN = 128
H = 64

NPAD = 10240
NW = 32
RANGE = NPAD // NW
ECH = 4000
NCHUNK = E // ECH
ACC = ECH + 16
G = 128
CHK = 2048
NB = CHK // G
FD = 4
EREG = 325632
PCLS = 1024
CB = PCLS // NW
NCH = 256
NEG = -3.0e38

_mesh = plsc.VectorSubcoreMesh(core_axis_name="c", subcore_axis_name="s")
_sc_params = pltpu.CompilerParams(needs_layout_passes=False)


def _wid():
    return lax.axis_index("s") * 2 + lax.axis_index("c")


def _popcount(m):
    return plsc.all_reduce_population_count(m)[0]


@functools.partial(
    pl.kernel,
    out_type=[
        jax.ShapeDtypeStruct((NW * EREG,), jnp.int32),
        jax.ShapeDtypeStruct((NW * EREG,), jnp.int32),
        jax.ShapeDtypeStruct((NW * 16,), jnp.int32),
    ],
    mesh=_mesh,
    compiler_params=_sc_params,
    scratch_types=[
        pltpu.VMEM((ECH,), jnp.int32),
        pltpu.VMEM((ECH,), jnp.int32),
        pltpu.VMEM((ECH,), jnp.int32),
        pltpu.VMEM((ECH,), jnp.int32),
        pltpu.VMEM((ACC,), jnp.int32),
        pltpu.VMEM((ACC,), jnp.int32),
        pltpu.VMEM((16,), jnp.int32),
        pltpu.SemaphoreType.DMA,
        pltpu.SemaphoreType.DMA,
        pltpu.SemaphoreType.DMA,
        pltpu.SemaphoreType.DMA,
    ],
)
def _partition(src_hbm, dst_hbm, psrc, pdst, cnts, sbuf0, sbuf1, dbuf0,
               dbuf1, acc_s, acc_d, cntbuf, semS0, semS1, semD0, semD1):
    w = _wid()
    lo = w * RANGE
    hi = lo + RANGE
    rbase = pl.multiple_of(w * EREG, 128)
    zeros16 = jnp.zeros((16,), jnp.int32)
    dummy16 = jnp.full((16,), RANGE, jnp.int32)
    semS = (semS0, semS1)
    semD = (semD0, semD1)
    sbuf = (sbuf0, sbuf1)
    dbuf = (dbuf0, dbuf1)

    def fire(c, sl):
        base = pl.multiple_of(c * ECH, ECH)
        pltpu.make_async_copy(src_hbm.at[pl.ds(base, ECH)], sbuf[sl],
                              semS[sl]).start()
        pltpu.make_async_copy(dst_hbm.at[pl.ds(base, ECH)], dbuf[sl],
                              semD[sl]).start()

    fire(0, 0)
    fire(1, 1)

    def pair(pi, tail):
        for sl in (0, 1):
            c = pi * 2 + sl
            pltpu.make_async_copy(src_hbm.at[pl.ds(0, ECH)], sbuf[sl],
                                  semS[sl]).wait()
            pltpu.make_async_copy(dst_hbm.at[pl.ds(0, ECH)], dbuf[sl],
                                  semD[sl]).wait()

            def vstep(v, k):
                d = dbuf[sl][pl.ds(v * 16, 16)]
                s = sbuf[sl][pl.ds(v * 16, 16)]
                m = (d >= lo) & (d < hi)
                plsc.store_compressed(acc_s.at[pl.ds(k, 16)], s, mask=m)
                plsc.store_compressed(acc_d.at[pl.ds(k, 16)], d - lo, mask=m)
                return k + _popcount(m)

            k = lax.fori_loop(0, ECH // 16, vstep, 0)

            @pl.when(c + 2 < NCHUNK)
            def _():
                fire(c + 2, sl)

            acc_s[pl.ds(k, 16)] = zeros16
            acc_d[pl.ds(k, 16)] = dummy16
            k16 = (k + 15) & ~15
            t = pl.multiple_of(rbase + tail, 16)
            pltpu.sync_copy(acc_s, psrc.at[pl.ds(t, ACC)])
            pltpu.sync_copy(acc_d, pdst.at[pl.ds(t, ACC)])
            tail = tail + k16
        return tail

    tail = lax.fori_loop(0, NCHUNK // 2, pair, 0)
    for j in range(CHK // 16):
        acc_s[pl.ds(j * 16, 16)] = zeros16
        acc_d[pl.ds(j * 16, 16)] = dummy16
    t = pl.multiple_of(rbase + tail, 16)
    pltpu.sync_copy(acc_s.at[pl.ds(0, CHK)], psrc.at[pl.ds(t, CHK)])
    pltpu.sync_copy(acc_d.at[pl.ds(0, CHK)], pdst.at[pl.ds(t, CHK)])
    n1k = (tail + CHK - 1) // CHK
    cntbuf[pl.ds(0, 16)] = jnp.full((16,), n1k, jnp.int32)
    pltpu.sync_copy(cntbuf, cnts.at[pl.ds(pl.multiple_of(w * 16, 16), 16)])


@functools.partial(
    pl.kernel,
    out_type=jax.ShapeDtypeStruct((NPAD, 2 * H), jnp.float32),
    mesh=_mesh,
    compiler_params=_sc_params,
    scratch_types=[
        pltpu.VMEM((16,), jnp.int32),
        pltpu.VMEM((CHK,), jnp.int32),
        pltpu.VMEM((CHK,), jnp.int32),
        pltpu.VMEM((CHK,), jnp.int32),
        pltpu.VMEM((CHK,), jnp.int32),
        pltpu.VMEM((G, 2 * H), jnp.float32),
        pltpu.VMEM((G, 2 * H), jnp.float32),
        pltpu.VMEM((G, 2 * H), jnp.float32),
        pltpu.VMEM((G, 2 * H), jnp.float32),
        pltpu.VMEM((RANGE + 1, 2 * H), jnp.float32),
        pltpu.SemaphoreType.DMA,
        pltpu.SemaphoreType.DMA,
        pltpu.SemaphoreType.DMA,
        pltpu.SemaphoreType.DMA,
        pltpu.SemaphoreType.DMA,
        pltpu.SemaphoreType.DMA,
        pltpu.SemaphoreType.DMA,
        pltpu.SemaphoreType.DMA,
    ],
)
def _aggregate(h_hbm, psrc, pdst, cnts, aggr_hbm, cntbuf, sidx0, sidx1,
               doff0, doff1, rowbuf0, rowbuf1, rowbuf2, rowbuf3, aggrbuf,
               semS0, semS1, semD0, semD1, semG0, semG1, semG2, semG3):
    w = _wid()
    lo = pl.multiple_of(w * RANGE, 8)
    rbase = pl.multiple_of(w * EREG, 128)
    pltpu.sync_copy(cnts.at[pl.ds(pl.multiple_of(w * 16, 16), 16)], cntbuf)
    pltpu.sync_copy(h_hbm.at[pl.ds(lo, RANGE)], aggrbuf.at[pl.ds(0, RANGE)])
    n1k = cntbuf[pl.ds(0, 16)][0]
    semS = (semS0, semS1)
    semD = (semD0, semD1)
    semG = (semG0, semG1, semG2, semG3)
    sidx = (sidx0, sidx1)
    doff = (doff0, doff1)
    rowbuf = (rowbuf0, rowbuf1, rowbuf2, rowbuf3)

    def fire_edges(c, sl):
        b = pl.multiple_of(rbase + c * CHK, CHK)
        pltpu.make_async_copy(psrc.at[pl.ds(b, CHK)], sidx[sl],
                              semS[sl]).start()
        pltpu.make_async_copy(pdst.at[pl.ds(b, CHK)], doff[sl],
                              semD[sl]).start()

    def fire_gather(sl, b, gsl):
        pltpu.make_async_copy(h_hbm.at[sidx[sl].at[pl.ds(b * G, G)]],
                              rowbuf[gsl], semG[gsl]).start()

    def wait_gather(gsl):
        pltpu.make_async_copy(h_hbm.at[pl.ds(0, G)], rowbuf[gsl],
                              semG[gsl]).wait()

    @pl.when(n1k > 0)
    def _():
        fire_edges(0, 0)

    @pl.when(n1k > 1)
    def _():
        fire_edges(1, 1)

    def pair(pi, _):
        for sl in (0, 1):
            c = pi * 2 + sl

            @pl.when(c < n1k)
            def _():
                pltpu.make_async_copy(psrc.at[pl.ds(0, CHK)], sidx[sl],
                                      semS[sl]).wait()
                pltpu.make_async_copy(pdst.at[pl.ds(0, CHK)], doff[sl],
                                      semD[sl]).wait()
                for b in range(FD):
                    fire_gather(sl, b, b)
                for b in range(NB):
                    gsl = b % FD
                    wait_gather(gsl)

                    def group(g, _):
                        dvec = doff[sl][pl.ds(b * G + g * 16, 16)]
                        for l in range(16):
                            off = dvec[l]
                            e = g * 16 + l
                            for j in range(H // 16):
                                sl2 = pl.ds(j * 16, 16)
                                aggrbuf[off, sl2] = jnp.maximum(
                                    aggrbuf[off, sl2],
                                    rowbuf[gsl][e, sl2])
                        return 0

                    lax.fori_loop(0, G // 16, group, 0)
                    if b + FD < NB:
                        fire_gather(sl, b + FD, gsl)

                @pl.when(c + 2 < n1k)
                def _():
                    fire_edges(c + 2, sl)
        return 0

    lax.fori_loop(0, (n1k + 1) // 2, pair, 0)
    pltpu.sync_copy(aggrbuf.at[pl.ds(0, RANGE)], aggr_hbm.at[pl.ds(lo, RANGE)])


@functools.partial(
    pl.kernel,
    out_type=jax.ShapeDtypeStruct((PCLS, 4 * H), jnp.float32),
    mesh=_mesh,
    compiler_params=_sc_params,
    scratch_types=[
        pltpu.VMEM((NPAD,), jnp.int32),
        pltpu.VMEM((NCH, 2 * H), jnp.float32),
        pltpu.VMEM((NCH, 2 * H), jnp.float32),
        pltpu.VMEM((CB, 2 * H), jnp.float32),
        pltpu.VMEM((CB, 2 * H), jnp.float32),
        pltpu.VMEM((CB, 16), jnp.float32),
        pltpu.VMEM((CB, 4 * H), jnp.float32),
    ],
)
def _pool(cl_hbm, h_hbm, aggr_hbm, out_hbm, clbuf, hbuf, abuf, maxacc, sumacc,
          cntacc, outbuf):
    w = _wid()
    clo = pl.multiple_of(w * CB, 8)
    pltpu.sync_copy(cl_hbm, clbuf)

    def scan(v, carry):
        nlo, nhi = carry
        cv = clbuf[pl.ds(v * 16, 16)]
        return (nlo + _popcount(cv < clo), nhi + _popcount(cv < clo + CB))

    nlo, nhi = lax.fori_loop(0, NPAD // 16, scan, (0, 0))

    def initrow(r, _):
        for j in range(2 * H // 16):
            maxacc[r, pl.ds(j * 16, 16)] = jnp.full((16,), NEG, jnp.float32)
            sumacc[r, pl.ds(j * 16, 16)] = jnp.zeros((16,), jnp.float32)
        cntacc[r, pl.ds(0, 16)] = jnp.zeros((16,), jnp.float32)
        return 0

    lax.fori_loop(0, CB, initrow, 0)

    def chunk(k, _):
        base = pl.multiple_of(k * NCH, NCH)

        @pl.when((base < nhi) & (base + NCH > nlo))
        def _():
            pltpu.sync_copy(h_hbm.at[pl.ds(base, NCH)], hbuf)
            pltpu.sync_copy(aggr_hbm.at[pl.ds(base, NCH)], abuf)
            e0 = jnp.maximum(nlo - base, 0)
            e1 = jnp.minimum(nhi - base, NCH)

            def group(g, _):
                cvec = clbuf[pl.ds(base + g * 16, 16)] - clo
                for l in range(16):
                    e = g * 16 + l

                    @pl.when((e >= e0) & (e < e1))
                    def _():
                        cc = cvec[l]
                        for j in range(H // 16):
                            sl = pl.ds(j * 16, 16)
                            sh = pl.ds(H + j * 16, 16)
                            hv = hbuf[e, sl]
                            av = abuf[e, sl]
                            maxacc[cc, sl] = jnp.maximum(maxacc[cc, sl], hv)
                            maxacc[cc, sh] = jnp.maximum(maxacc[cc, sh], av)
                            sumacc[cc, sl] = sumacc[cc, sl] + hv
                            sumacc[cc, sh] = sumacc[cc, sh] + av
                        cnt = cntacc[cc, pl.ds(0, 16)]
                        cntacc[cc, pl.ds(0, 16)] = cnt + 1.0
                return 0

            lax.fori_loop(e0 // 16, (e1 + 15) // 16, group, 0)
        return 0

    lax.fori_loop(0, NPAD // NCH, chunk, 0)

    def fin(r, _):
        cnt = cntacc[r, pl.ds(0, 16)]
        nonempty = cnt > 0.0
        den = jnp.maximum(cnt, 1.0)
        for j in range(2 * H // 16):
            sl = pl.ds(j * 16, 16)
            outbuf[r, sl] = jnp.where(nonempty, maxacc[r, sl], 0.0)
            outbuf[r, pl.ds(2 * H + j * 16, 16)] = sumacc[r, sl] / den
        return 0

    lax.fori_loop(0, CB, fin, 0)
    pltpu.sync_copy(outbuf, out_hbm.at[pl.ds(clo, CB)])


def _ln_tc(h, g, b):
    mu = jnp.mean(h, axis=-1, keepdims=True)
    v = jnp.mean((h - mu) * (h - mu), axis=-1, keepdims=True)
    return (h - mu) * lax.rsqrt(v + 1e-5) * g + b


def _mm(a, b):
    return lax.dot_general(a, b, (((1,), (0,)), ((), ())),
                           preferred_element_type=jnp.float32)


def _dense_body(x_ref, w1, b1, g1, be1, w2, b2, g2, be2, wr, br, gr, ber,
                h_ref):
    x = x_ref[...]
    a = _ln_tc(_mm(x, w1[...]) + b1[...], g1[...], be1[...])
    a = jnp.maximum(a, 0.0)
    t = _ln_tc(_mm(a, w2[...]) + b2[...], g2[...], be2[...])
    r = _ln_tc(_mm(x, wr[...]) + br[...], gr[...], ber[...])
    h = jnp.maximum(t + r, 0.0)
    h_ref[...] = jnp.concatenate([h, jnp.zeros_like(h)], axis=1)


def _dense(xin, p):
    d = xin.shape[1]
    blk = 512
    vec = lambda a: a.reshape(1, H)
    wspec = lambda s: pl.BlockSpec(s, lambda i: (0, 0))
    return pl.pallas_call(
        _dense_body,
        grid=(NPAD // blk,),
        in_specs=[
            pl.BlockSpec((blk, d), lambda i: (i, 0)),
            wspec((d, H)), wspec((1, H)), wspec((1, H)), wspec((1, H)),
            wspec((H, H)), wspec((1, H)), wspec((1, H)), wspec((1, H)),
            wspec((d, H)), wspec((1, H)), wspec((1, H)), wspec((1, H)),
        ],
        out_specs=pl.BlockSpec((blk, 2 * H), lambda i: (i, 0)),
        out_shape=jax.ShapeDtypeStruct((NPAD, 2 * H), jnp.float32),
    )(xin, p['w1'], vec(p['b1']), vec(p['g1']), vec(p['be1']),
      p['w2'], vec(p['b2']), vec(p['g2']), vec(p['be2']),
      p['wr'], vec(p['br']), vec(p['gr']), vec(p['ber']))


def _final_body(p0, p1, p2, w, b, o_ref):
    xs = p0[...] + p1[...] + p2[...]
    out = jnp.maximum(_mm(xs, w[...]) + b[...], 0.0)
    row = lax.broadcasted_iota(jnp.int32, (PCLS, 1), 0)
    out = jnp.where(row < C, out, 0.0)
    nrm = jnp.sqrt(jnp.sum(out * out, axis=0, keepdims=True))
    o_ref[...] = out / (nrm + 1e-8)


def _final(p0, p1, p2, w, b):
    return pl.pallas_call(
        _final_body,
        out_shape=jax.ShapeDtypeStruct((PCLS, 2 * H), jnp.float32),
    )(p0, p1, p2, w, b.reshape(1, 2 * H))


def kernel(x, edge_index, cluster, time_step_len, params):
    src = edge_index[0]
    dst = edge_index[1]
    x_pad = jnp.zeros((NPAD, IN), jnp.float32).at[:N].set(x)
    cl_pad = jnp.full((NPAD,), PCLS - 1, jnp.int32).at[:N].set(cluster)

    psrc, pdst, cnts = _partition(src, dst)

    xin = x_pad
    pools = []
    for li in range(3):
        p = params['l%d' % li]
        h = _dense(xin, p)
        aggr = _aggregate(h, psrc, pdst, cnts)
        pools.append(_pool(cl_pad, h, aggr))
        if li < 2:
            xin = jnp.concatenate([h[:, :H], aggr[:, :H]], axis=1)

    out = _final(pools[0], pools[1], pools[2], params['wout'], params['bout'])
    return out[:C]


# speedup vs baseline: 5.4861x; 5.4861x over previous
import functools

import jax
import jax.numpy as jnp
from jax import lax
from jax.experimental import pallas as pl
from jax.experimental.pallas import tpu as pltpu
from jax.experimental.pallas import tpu_sc as plsc

N = 10000
E = 320000
C = 1000
IN = 128
H = 64

NPAD = 10240
NW = 32
RANGE = NPAD // NW
ECH = 4000
NCHUNK = E // ECH
ACC = ECH + 16
EREG = 325632
BCH = 256
SCH = 256
NBK = NPAD // SCH
BACC = 528
BKCAP = E + 8192
ECH3 = 1024
PCLS = 1024
CB = PCLS // NW
NCH = 256
NEG = -3.0e38

_mesh = plsc.VectorSubcoreMesh(core_axis_name="c", subcore_axis_name="s")
_sc_params = pltpu.CompilerParams(needs_layout_passes=False)


def _wid():
    return lax.axis_index("s") * 2 + lax.axis_index("c")


def _popcount(m):
    return plsc.all_reduce_population_count(m)[0]


@functools.partial(
    pl.kernel,
    out_type=[
        jax.ShapeDtypeStruct((NW * EREG,), jnp.int32),
        jax.ShapeDtypeStruct((NW * EREG,), jnp.int32),
        jax.ShapeDtypeStruct((NW * 16,), jnp.int32),
    ],
    mesh=_mesh,
    compiler_params=_sc_params,
    scratch_types=[
        pltpu.VMEM((ECH,), jnp.int32),
        pltpu.VMEM((ECH,), jnp.int32),
        pltpu.VMEM((ECH,), jnp.int32),
        pltpu.VMEM((ECH,), jnp.int32),
        pltpu.VMEM((ACC,), jnp.int32),
        pltpu.VMEM((ACC,), jnp.int32),
        pltpu.VMEM((16,), jnp.int32),
        pltpu.SemaphoreType.DMA,
        pltpu.SemaphoreType.DMA,
        pltpu.SemaphoreType.DMA,
        pltpu.SemaphoreType.DMA,
    ],
)
def _partition(src_hbm, dst_hbm, psrc, pdst, cnts, sbuf0, sbuf1, dbuf0,
               dbuf1, acc_s, acc_d, cntbuf, semS0, semS1, semD0, semD1):
    w = _wid()
    lo = w * RANGE
    hi = lo + RANGE
    rbase = pl.multiple_of(w * EREG, 128)
    zeros16 = jnp.zeros((16,), jnp.int32)
    dummy16 = jnp.full((16,), RANGE, jnp.int32)
    semS = (semS0, semS1)
    semD = (semD0, semD1)
    sbuf = (sbuf0, sbuf1)
    dbuf = (dbuf0, dbuf1)

    def fire(c, sl):
        base = pl.multiple_of(c * ECH, 8)
        pltpu.make_async_copy(src_hbm.at[pl.ds(base, ECH)], sbuf[sl],
                              semS[sl]).start()
        pltpu.make_async_copy(dst_hbm.at[pl.ds(base, ECH)], dbuf[sl],
                              semD[sl]).start()

    fire(0, 0)
    fire(1, 1)

    def pair(pi, tail):
        for sl in (0, 1):
            c = pi * 2 + sl
            pltpu.make_async_copy(src_hbm.at[pl.ds(0, ECH)], sbuf[sl],
                                  semS[sl]).wait()
            pltpu.make_async_copy(dst_hbm.at[pl.ds(0, ECH)], dbuf[sl],
                                  semD[sl]).wait()

            def vstep(v, k):
                d = dbuf[sl][pl.ds(v * 16, 16)]
                s = sbuf[sl][pl.ds(v * 16, 16)]
                m = (d >= lo) & (d < hi)
                plsc.store_compressed(acc_s.at[pl.ds(k, 16)], s, mask=m)
                plsc.store_compressed(acc_d.at[pl.ds(k, 16)], d - lo, mask=m)
                return k + _popcount(m)

            k = lax.fori_loop(0, ECH // 16, vstep, 0)

            @pl.when(c + 2 < NCHUNK)
            def _():
                fire(c + 2, sl)

            acc_s[pl.ds(k, 16)] = zeros16
            acc_d[pl.ds(k, 16)] = dummy16
            k16 = (k + 15) & ~15
            t = pl.multiple_of(rbase + tail, 16)
            pltpu.sync_copy(acc_s, psrc.at[pl.ds(t, ACC)])
            pltpu.sync_copy(acc_d, pdst.at[pl.ds(t, ACC)])
            tail = tail + k16
        return tail

    tail = lax.fori_loop(0, NCHUNK // 2, pair, 0)
    for j in range(BCH // 16):
        acc_s[pl.ds(j * 16, 16)] = zeros16
        acc_d[pl.ds(j * 16, 16)] = dummy16
    t = pl.multiple_of(rbase + tail, 16)
    pltpu.sync_copy(acc_s.at[pl.ds(0, BCH)], psrc.at[pl.ds(t, BCH)])
    pltpu.sync_copy(acc_d.at[pl.ds(0, BCH)], pdst.at[pl.ds(t, BCH)])
    n512 = (tail + BCH - 1) // BCH
    cntbuf[pl.ds(0, 16)] = jnp.full((16,), n512, jnp.int32)
    pltpu.sync_copy(cntbuf, cnts.at[pl.ds(pl.multiple_of(w * 16, 16), 16)])


@functools.partial(
    pl.kernel,
    out_type=[
        jax.ShapeDtypeStruct((NW * NBK * BKCAP,), jnp.int32),
        jax.ShapeDtypeStruct((NW * NBK * BKCAP,), jnp.int32),
        jax.ShapeDtypeStruct((NW * 1024,), jnp.int32),
    ],
    mesh=_mesh,
    compiler_params=_sc_params,
    scratch_types=[
        pltpu.VMEM((BCH,), jnp.int32),
        pltpu.VMEM((BCH,), jnp.int32),
        pltpu.VMEM((BCH,), jnp.int32),
        pltpu.VMEM((BCH,), jnp.int32),
        pltpu.VMEM((NBK * BACC,), jnp.int32),
        pltpu.VMEM((NBK * BACC,), jnp.int32),
        pltpu.VMEM((1024,), jnp.int32),
        pltpu.SemaphoreType.DMA,
        pltpu.SemaphoreType.DMA,
        pltpu.SemaphoreType.DMA,
        pltpu.SemaphoreType.DMA,
    ],
)
def _bucket(psrc, pdst, cnts, bsrc, bdst, bmeta, sbuf0, sbuf1, dbuf0, dbuf1,
            acc_s, acc_d, cmeta, semS0, semS1, semD0, semD1):
    w = _wid()
    rbase = pl.multiple_of(w * EREG, 128)
    zeros16 = jnp.zeros((16,), jnp.int32)
    dummy16 = jnp.full((16,), RANGE, jnp.int32)
    semS = (semS0, semS1)
    semD = (semD0, semD1)
    sbuf = (sbuf0, sbuf1)
    dbuf = (dbuf0, dbuf1)

    pltpu.sync_copy(cnts.at[pl.ds(pl.multiple_of(w * 16, 16), 16)],
                    cmeta.at[pl.ds(0, 16)])
    n512 = cmeta[pl.ds(0, 16)][0]

    def fire(c, sl):
        b = pl.multiple_of(rbase + c * BCH, 8)
        pltpu.make_async_copy(psrc.at[pl.ds(b, BCH)], sbuf[sl],
                              semS[sl]).start()
        pltpu.make_async_copy(pdst.at[pl.ds(b, BCH)], dbuf[sl],
                              semD[sl]).start()

    @pl.when(n512 > 0)
    def _():
        fire(0, 0)

    @pl.when(n512 > 1)
    def _():
        fire(1, 1)

    def flush(k, cnt, tail):
        acc_s[pl.ds(k * BACC + cnt, 16)] = zeros16
        acc_d[pl.ds(k * BACC + cnt, 16)] = dummy16
        t = pl.multiple_of((w * NBK + k) * BKCAP + tail, 16)
        pltpu.sync_copy(acc_s.at[pl.ds(k * BACC, BACC)],
                        bsrc.at[pl.ds(t, BACC)])
        pltpu.sync_copy(acc_d.at[pl.ds(k * BACC, BACC)],
                        bdst.at[pl.ds(t, BACC)])
        return tail + ((cnt + 15) & ~15)

    def pair(pi, carry):
        for sl in (0, 1):
            c = pi * 2 + sl

            def proc(carry=carry, c=c, sl=sl):
                pltpu.make_async_copy(psrc.at[pl.ds(0, BCH)], sbuf[sl],
                                      semS[sl]).wait()
                pltpu.make_async_copy(pdst.at[pl.ds(0, BCH)], dbuf[sl],
                                      semD[sl]).wait()

                out = []
                for k in range(NBK):
                    cnt, tail = carry[2 * k], carry[2 * k + 1]

                    def bstep(v, ck, k=k, sl=sl):
                        s = sbuf[sl][pl.ds(v * 16, 16)]
                        d = dbuf[sl][pl.ds(v * 16, 16)]
                        m = (s >= k * SCH) & (s < (k + 1) * SCH)
                        plsc.store_compressed(
                            acc_s.at[pl.ds(k * BACC + ck, 16)],
                            s - k * SCH, mask=m)
                        plsc.store_compressed(
                            acc_d.at[pl.ds(k * BACC + ck, 16)], d, mask=m)
                        return ck + _popcount(m)

                    cnt = lax.fori_loop(0, BCH // 16, bstep, cnt)
                    cnt, tail = lax.cond(
                        cnt >= BCH,
                        lambda cnt=cnt, tail=tail, k=k: (0, flush(k, cnt,
                                                                  tail)),
                        lambda cnt=cnt, tail=tail: (cnt, tail))
                    out += [cnt, tail]

                @pl.when(c + 2 < n512)
                def _():
                    fire(c + 2, sl)
                return tuple(out)

            carry = lax.cond(c < n512, proc, lambda carry=carry: carry)
        return carry

    carry = lax.fori_loop(0, (n512 + 1) // 2, pair, (0,) * (2 * NBK))
    for k in range(NBK):
        tail = flush(k, carry[2 * k], carry[2 * k + 1])
        cmeta[pl.ds(k * 16, 16)] = jnp.full((16,), tail, jnp.int32)
    pltpu.sync_copy(cmeta, bmeta.at[pl.ds(pl.multiple_of(w * 1024, 1024), 1024)])


@functools.partial(
    pl.kernel,
    out_type=jax.ShapeDtypeStruct((NPAD, H), jnp.float32),
    mesh=_mesh,
    compiler_params=_sc_params,
    scratch_types=[
        pltpu.VMEM((1024,), jnp.int32),
        pltpu.VMEM((SCH, H), jnp.float32),
        pltpu.VMEM((SCH, H), jnp.float32),
        pltpu.VMEM((ECH3,), jnp.int32),
        pltpu.VMEM((ECH3,), jnp.int32),
        pltpu.VMEM((ECH3,), jnp.int32),
        pltpu.VMEM((ECH3,), jnp.int32),
        pltpu.VMEM((RANGE + 1, H), jnp.float32),
        pltpu.SemaphoreType.DMA,
        pltpu.SemaphoreType.DMA,
        pltpu.SemaphoreType.DMA,
        pltpu.SemaphoreType.DMA,
        pltpu.SemaphoreType.DMA,
        pltpu.SemaphoreType.DMA,
    ],
)
def _aggregate(h_hbm, bsrc, bdst, bmeta, aggr_hbm, metab, hb0, hb1,
               es0, es1, ed0, ed1, aggrbuf,
               semH0, semH1, semS0, semS1, semD0, semD1):
    w = _wid()
    lo = pl.multiple_of(w * RANGE, 8)
    pltpu.sync_copy(bmeta.at[pl.ds(pl.multiple_of(w * 1024, 1024), 1024)],
                    metab)
    pltpu.sync_copy(h_hbm.at[pl.ds(lo, RANGE)], aggrbuf.at[pl.ds(0, RANGE)])
    semH = (semH0, semH1)
    semS = (semS0, semS1)
    semD = (semD0, semD1)
    hb = (hb0, hb1)
    es = (es0, es1)
    ed = (ed0, ed1)

    def fire_h(k, sl):
        b = pl.multiple_of(k * SCH, 8)
        pltpu.make_async_copy(h_hbm.at[pl.ds(b, SCH)], hb[sl],
                              semH[sl]).start()

    def fire_e(k, sl):
        b = pl.multiple_of((w * NBK + k) * BKCAP, 8)
        pltpu.make_async_copy(bsrc.at[pl.ds(b, ECH3)], es[sl],
                              semS[sl]).start()
        pltpu.make_async_copy(bdst.at[pl.ds(b, ECH3)], ed[sl],
                              semD[sl]).start()

    fire_h(0, 0)
    fire_e(0, 0)

    def process(sl, ngroups):
        def group(g, _):
            svec = es[sl][pl.ds(g * 16, 16)]
            dvec = ed[sl][pl.ds(g * 16, 16)]
            for l in range(16):
                soff = svec[l]
                off = dvec[l]
                for j in range(H // 16):
                    sl2 = pl.ds(j * 16, 16)
                    aggrbuf[off, sl2] = jnp.maximum(aggrbuf[off, sl2],
                                                    hb[sl][soff, sl2])
            return 0

        lax.fori_loop(0, ngroups, group, 0)

    def pair(pi, _):
        for sl in (0, 1):
            k = pi * 2 + sl
            pltpu.make_async_copy(h_hbm.at[pl.ds(0, SCH)], hb[sl],
                                  semH[sl]).wait()
            pltpu.make_async_copy(bsrc.at[pl.ds(0, ECH3)], es[sl],
                                  semS[sl]).wait()
            pltpu.make_async_copy(bdst.at[pl.ds(0, ECH3)], ed[sl],
                                  semD[sl]).wait()

            @pl.when(k + 1 < NBK)
            def _():
                fire_h(k + 1, 1 - sl)
                fire_e(k + 1, 1 - sl)

            cntk = metab[pl.ds(k * 16, 16)][0]
            process(sl, jnp.minimum(cntk, ECH3) // 16)

            def extra(sc, _, k=k, sl=sl, cntk=cntk):
                b = pl.multiple_of((w * NBK + k) * BKCAP, 8) + sc * ECH3
                pltpu.sync_copy(bsrc.at[pl.ds(b, ECH3)], es[sl])
                pltpu.sync_copy(bdst.at[pl.ds(b, ECH3)], ed[sl])
                process(sl, jnp.minimum(cntk - sc * ECH3, ECH3) // 16)
                return 0

            lax.fori_loop(1, (cntk + ECH3 - 1) // ECH3, extra, 0)
        return 0

    lax.fori_loop(0, NBK // 2, pair, 0)
    pltpu.sync_copy(aggrbuf.at[pl.ds(0, RANGE)], aggr_hbm.at[pl.ds(lo, RANGE)])


@functools.partial(
    pl.kernel,
    out_type=jax.ShapeDtypeStruct((PCLS, 4 * H), jnp.float32),
    mesh=_mesh,
    compiler_params=_sc_params,
    scratch_types=[
        pltpu.VMEM((NPAD,), jnp.int32),
        pltpu.VMEM((NCH, H), jnp.float32),
        pltpu.VMEM((NCH, H), jnp.float32),
        pltpu.VMEM((CB, 2 * H), jnp.float32),
        pltpu.VMEM((CB, 2 * H), jnp.float32),
        pltpu.VMEM((CB, 16), jnp.float32),
        pltpu.VMEM((CB, 4 * H), jnp.float32),
    ],
)
def _pool(cl_hbm, h_hbm, aggr_hbm, out_hbm, clbuf, hbuf, abuf, maxacc, sumacc,
          cntacc, outbuf):
    w = _wid()
    clo = pl.multiple_of(w * CB, 8)
    pltpu.sync_copy(cl_hbm, clbuf)

    def scan(v, carry):
        nlo, nhi = carry
        cv = clbuf[pl.ds(v * 16, 16)]
        return (nlo + _popcount(cv < clo), nhi + _popcount(cv < clo + CB))

    nlo, nhi = lax.fori_loop(0, NPAD // 16, scan, (0, 0))

    def initrow(r, _):
        for j in range(2 * H // 16):
            maxacc[r, pl.ds(j * 16, 16)] = jnp.full((16,), NEG, jnp.float32)
            sumacc[r, pl.ds(j * 16, 16)] = jnp.zeros((16,), jnp.float32)
        cntacc[r, pl.ds(0, 16)] = jnp.zeros((16,), jnp.float32)
        return 0

    lax.fori_loop(0, CB, initrow, 0)

    def chunk(kk, _):
        base = pl.multiple_of(kk * NCH, NCH)

        @pl.when((base < nhi) & (base + NCH > nlo))
        def _():
            pltpu.sync_copy(h_hbm.at[pl.ds(base, NCH)], hbuf)
            pltpu.sync_copy(aggr_hbm.at[pl.ds(base, NCH)], abuf)
            e0 = jnp.maximum(nlo - base, 0)
            e1 = jnp.minimum(nhi - base, NCH)

            def group(g, _):
                cvec = clbuf[pl.ds(base + g * 16, 16)] - clo
                for l in range(16):
                    e = g * 16 + l

                    @pl.when((e >= e0) & (e < e1))
                    def _():
                        cc = cvec[l]
                        for j in range(H // 16):
                            sl = pl.ds(j * 16, 16)
                            sh = pl.ds(H + j * 16, 16)
                            hv = hbuf[e, sl]
                            av = abuf[e, sl]
                            maxacc[cc, sl] = jnp.maximum(maxacc[cc, sl], hv)
                            maxacc[cc, sh] = jnp.maximum(maxacc[cc, sh], av)
                            sumacc[cc, sl] = sumacc[cc, sl] + hv
                            sumacc[cc, sh] = sumacc[cc, sh] + av
                        cnt = cntacc[cc, pl.ds(0, 16)]
                        cntacc[cc, pl.ds(0, 16)] = cnt + 1.0
                return 0

            lax.fori_loop(e0 // 16, (e1 + 15) // 16, group, 0)
        return 0

    lax.fori_loop(0, NPAD // NCH, chunk, 0)

    def fin(r, _):
        cnt = cntacc[r, pl.ds(0, 16)]
        nonempty = cnt > 0.0
        den = jnp.maximum(cnt, 1.0)
        for j in range(2 * H // 16):
            sl = pl.ds(j * 16, 16)
            outbuf[r, sl] = jnp.where(nonempty, maxacc[r, sl], 0.0)
            outbuf[r, pl.ds(2 * H + j * 16, 16)] = sumacc[r, sl] / den
        return 0

    lax.fori_loop(0, CB, fin, 0)
    pltpu.sync_copy(outbuf, out_hbm.at[pl.ds(clo, CB)])


def _ln_tc(h, g, b):
    mu = jnp.mean(h, axis=-1, keepdims=True)
    v = jnp.mean((h - mu) * (h - mu), axis=-1, keepdims=True)
    return (h - mu) * lax.rsqrt(v + 1e-5) * g + b


def _mm(a, b):
    return lax.dot_general(a, b, (((1,), (0,)), ((), ())),
                           preferred_element_type=jnp.float32)


def _dense_body(x_ref, w1, b1, g1, be1, w2, b2, g2, be2, wr, br, gr, ber,
                h_ref):
    x = x_ref[...]
    a = _ln_tc(_mm(x, w1[...]) + b1[...], g1[...], be1[...])
    a = jnp.maximum(a, 0.0)
    t = _ln_tc(_mm(a, w2[...]) + b2[...], g2[...], be2[...])
    r = _ln_tc(_mm(x, wr[...]) + br[...], gr[...], ber[...])
    h_ref[...] = jnp.maximum(t + r, 0.0)


def _dense(xin, p):
    d = xin.shape[1]
    blk = 512
    vec = lambda a: a.reshape(1, H)
    wspec = lambda s: pl.BlockSpec(s, lambda i: (0, 0))
    return pl.pallas_call(
        _dense_body,
        grid=(NPAD // blk,),
        in_specs=[
            pl.BlockSpec((blk, d), lambda i: (i, 0)),
            wspec((d, H)), wspec((1, H)), wspec((1, H)), wspec((1, H)),
            wspec((H, H)), wspec((1, H)), wspec((1, H)), wspec((1, H)),
            wspec((d, H)), wspec((1, H)), wspec((1, H)), wspec((1, H)),
        ],
        out_specs=pl.BlockSpec((blk, H), lambda i: (i, 0)),
        out_shape=jax.ShapeDtypeStruct((NPAD, H), jnp.float32),
    )(xin, p['w1'], vec(p['b1']), vec(p['g1']), vec(p['be1']),
      p['w2'], vec(p['b2']), vec(p['g2']), vec(p['be2']),
      p['wr'], vec(p['br']), vec(p['gr']), vec(p['ber']))


def _final_body(p0, p1, p2, w, b, o_ref):
    xs = p0[...] + p1[...] + p2[...]
    out = jnp.maximum(_mm(xs, w[...]) + b[...], 0.0)
    row = lax.broadcasted_iota(jnp.int32, (PCLS, 1), 0)
    out = jnp.where(row < C, out, 0.0)
    nrm = jnp.sqrt(jnp.sum(out * out, axis=0, keepdims=True))
    o_ref[...] = out / (nrm + 1e-8)


def _final(p0, p1, p2, w, b):
    return pl.pallas_call(
        _final_body,
        out_shape=jax.ShapeDtypeStruct((PCLS, 2 * H), jnp.float32),
    )(p0, p1, p2, w, b.reshape(1, 2 * H))


def kernel(x, edge_index, cluster, time_step_len, params):
    src = edge_index[0]
    dst = edge_index[1]
    x_pad = jnp.zeros((NPAD, IN), jnp.float32).at[:N].set(x)
    cl_pad = jnp.full((NPAD,), PCLS - 1, jnp.int32).at[:N].set(cluster)

    psrc, pdst, cnts = _partition(src, dst)
    bsrc, bdst, bmeta = _bucket(psrc, pdst, cnts)

    xin = x_pad
    pools = []
    for li in range(3):
        p = params['l%d' % li]
        h = _dense(xin, p)
        aggr = _aggregate(h, bsrc, bdst, bmeta)
        pools.append(_pool(cl_pad, h, aggr))
        if li < 2:
            xin = jnp.concatenate([h, aggr], axis=1)

    out = _final(pools[0], pools[1], pools[2], params['wout'], params['bout'])
    return out[:C]
